# Initial kernel scaffold; baseline (speedup 1.0000x reference)
import jax
import jax.numpy as jnp
from jax.experimental import pallas as pl


def kernel(positions, pe):
    raise NotImplementedError("write your pallas kernel here")


# speedup vs baseline: 3.7931x; 3.7931x over previous
import functools

import jax
import jax.numpy as jnp
from jax import lax
from jax.experimental import pallas as pl
from jax.experimental.pallas import tpu as pltpu
from jax.experimental.pallas import tpu_sc as plsc

D_MODEL = 128
CHUNK = 128


def _gather_sc(n_total: int):
    info = plsc.get_sparse_core_info()
    nw = info.num_cores * info.num_subcores
    assert n_total % (nw * CHUNK) == 0
    n_per_w = n_total // nw
    n_chunks = n_per_w // CHUNK

    mesh = plsc.VectorSubcoreMesh(core_axis_name="c", subcore_axis_name="s")

    @functools.partial(
        pl.kernel,
        mesh=mesh,
        out_type=jax.ShapeDtypeStruct((n_total, D_MODEL), jnp.float32),
        scratch_types=[
            pltpu.VMEM((CHUNK,), jnp.int32),
            pltpu.VMEM((CHUNK, D_MODEL), jnp.float32),
            pltpu.SemaphoreType.DMA,
        ],
    )
    def gather_kernel(idx_hbm, table_hbm, out_hbm, idx_v, rows_v, sem):
        wid = lax.axis_index("s") * info.num_cores + lax.axis_index("c")
        base = wid * n_per_w

        def body(i, carry):
            off = base + i * CHUNK
            pltpu.sync_copy(idx_hbm.at[pl.ds(off, CHUNK)], idx_v)
            pltpu.async_copy(table_hbm.at[idx_v], rows_v, sem).wait()
            pltpu.sync_copy(rows_v, out_hbm.at[pl.ds(off, CHUNK)])
            return carry

        lax.fori_loop(0, n_chunks, body, 0)

    return gather_kernel


def kernel(positions, pe):
    b, t = positions.shape
    n_total = b * t
    idx = positions.reshape(n_total).astype(jnp.int32)
    out = _gather_sc(n_total)(idx, pe)
    return out.reshape(b, t, D_MODEL)


# speedup vs baseline: 3.9661x; 1.0456x over previous
import functools

import jax
import jax.numpy as jnp
from jax import lax
from jax.experimental import pallas as pl
from jax.experimental.pallas import tpu as pltpu
from jax.experimental.pallas import tpu_sc as plsc

D_MODEL = 128
SUB = 128
KC = 2
ROWS = SUB * KC


def _gather_sc(n_total: int):
    info = plsc.get_sparse_core_info()
    nw = info.num_cores * info.num_subcores
    n_per_w = n_total // nw
    n_sub = n_per_w // SUB
    n_outer = n_per_w // ROWS
    assert n_total % (nw * ROWS) == 0

    mesh = plsc.VectorSubcoreMesh(core_axis_name="c", subcore_axis_name="s")

    @functools.partial(
        pl.kernel,
        mesh=mesh,
        out_type=jax.ShapeDtypeStruct((n_total, D_MODEL), jnp.float32),
        scratch_types=[
            pltpu.VMEM((n_sub, SUB), jnp.int32),
            pltpu.VMEM((ROWS, D_MODEL), jnp.float32),
            pltpu.VMEM((ROWS, D_MODEL), jnp.float32),
            pltpu.SemaphoreType.DMA,
            pltpu.SemaphoreType.DMA,
            pltpu.SemaphoreType.DMA,
            pltpu.SemaphoreType.DMA,
        ],
    )
    def gather_kernel(idx_hbm, table_hbm, out_hbm, idx_v, rows0, rows1,
                      gsem0, gsem1, ssem0, ssem1):
        wid = lax.axis_index("s") * info.num_cores + lax.axis_index("c")
        base = wid * n_per_w
        pltpu.sync_copy(idx_hbm.at[wid], idx_v)

        rows = (rows0, rows1)
        gsems = (gsem0, gsem1)
        ssems = (ssem0, ssem1)

        def body(i2, carry):
            for b in range(2):
                i = i2 * 2 + b
                buf, gsem, ssem = rows[b], gsems[b], ssems[b]

                @pl.when(i2 > 0)
                def _():
                    pltpu.make_async_copy(
                        buf, out_hbm.at[pl.ds(base, ROWS)], ssem).wait()

                descs = []
                for j in range(KC):
                    descs.append(pltpu.async_copy(
                        table_hbm.at[idx_v.at[i * KC + j]],
                        buf.at[pl.ds(j * SUB, SUB)],
                        gsem))
                for d in descs:
                    d.wait()
                pltpu.async_copy(
                    buf, out_hbm.at[pl.ds(base + i * ROWS, ROWS)], ssem)
            return carry

        lax.fori_loop(0, n_outer // 2, body, 0)

        for b in range(2):
            pltpu.make_async_copy(
                rows[b], out_hbm.at[pl.ds(base, ROWS)], ssems[b]).wait()

    return gather_kernel


def kernel(positions, pe):
    b, t = positions.shape
    n_total = b * t
    info = plsc.get_sparse_core_info()
    nw = info.num_cores * info.num_subcores
    idx = positions.reshape(nw, (n_total // nw) // SUB, SUB).astype(jnp.int32)
    out = _gather_sc(n_total)(idx, pe)
    return out.reshape(b, t, D_MODEL)


# speedup vs baseline: 15.5065x; 3.9098x over previous
import functools

import jax
import jax.numpy as jnp
from jax import lax
from jax.experimental import pallas as pl
from jax.experimental.pallas import tpu as pltpu
from jax.experimental.pallas import tpu_sc as plsc

D_MODEL = 128
SUB = 128
KC = 2
ROWS = SUB * KC


def _gather_sc(n_total: int):
    info = plsc.get_sparse_core_info()
    nw = info.num_cores * info.num_subcores
    n_per_w = n_total // nw
    n_sub = n_per_w // SUB
    n_outer = n_per_w // ROWS
    assert n_total % (nw * ROWS) == 0

    mesh = plsc.VectorSubcoreMesh(core_axis_name="c", subcore_axis_name="s")

    @functools.partial(
        pl.kernel,
        mesh=mesh,
        out_type=jax.ShapeDtypeStruct((n_total, D_MODEL), jnp.float32),
        scratch_types=[
            pltpu.VMEM((n_sub, SUB), jnp.int32),
            pltpu.VMEM((ROWS, D_MODEL), jnp.float32),
            pltpu.VMEM((ROWS, D_MODEL), jnp.float32),
            pltpu.VMEM_SHARED((366, D_MODEL), jnp.float32),
            pltpu.SemaphoreType.DMA,
            pltpu.SemaphoreType.DMA,
            pltpu.SemaphoreType.DMA,
            pltpu.SemaphoreType.DMA,
        ],
    )
    def gather_kernel(idx_hbm, table_hbm, out_hbm, idx_v, rows0, rows1,
                      table_sh, gsem0, gsem1, ssem0, ssem1):
        wid = lax.axis_index("s") * info.num_cores + lax.axis_index("c")
        base = wid * n_per_w

        @pl.when(lax.axis_index("s") == 0)
        def _():
            pltpu.sync_copy(table_hbm, table_sh)
        plsc.subcore_barrier()

        pltpu.sync_copy(idx_hbm.at[wid], idx_v)

        rows = (rows0, rows1)
        gsems = (gsem0, gsem1)
        ssems = (ssem0, ssem1)

        def body(i2, carry):
            for b in range(2):
                i = i2 * 2 + b
                buf, gsem, ssem = rows[b], gsems[b], ssems[b]

                @pl.when(i2 > 0)
                def _():
                    pltpu.make_async_copy(
                        buf, out_hbm.at[pl.ds(base, ROWS)], ssem).wait()

                descs = []
                for j in range(KC):
                    descs.append(pltpu.async_copy(
                        table_sh.at[idx_v.at[i * KC + j]],
                        buf.at[pl.ds(j * SUB, SUB)],
                        gsem))
                for d in descs:
                    d.wait()
                pltpu.async_copy(
                    buf, out_hbm.at[pl.ds(base + i * ROWS, ROWS)], ssem)
            return carry

        lax.fori_loop(0, n_outer // 2, body, 0)

        for b in range(2):
            pltpu.make_async_copy(
                rows[b], out_hbm.at[pl.ds(base, ROWS)], ssems[b]).wait()

    return gather_kernel


def kernel(positions, pe):
    b, t = positions.shape
    n_total = b * t
    info = plsc.get_sparse_core_info()
    nw = info.num_cores * info.num_subcores
    idx = positions.reshape(nw, (n_total // nw) // SUB, SUB).astype(jnp.int32)
    out = _gather_sc(n_total)(idx, pe)
    return out.reshape(b, t, D_MODEL)


# speedup vs baseline: 15.7354x; 1.0148x over previous
import functools

import jax
import jax.numpy as jnp
from jax import lax
from jax.experimental import pallas as pl
from jax.experimental.pallas import tpu as pltpu
from jax.experimental.pallas import tpu_sc as plsc

D_MODEL = 128
SUB = 128
NBUF = 4


def _gather_sc(n_total: int):
    info = plsc.get_sparse_core_info()
    nw = info.num_cores * info.num_subcores
    n_per_w = n_total // nw
    n_sub = n_per_w // SUB
    assert n_total % (nw * SUB * NBUF) == 0
    n_outer = n_sub // NBUF

    mesh = plsc.VectorSubcoreMesh(core_axis_name="c", subcore_axis_name="s")

    @functools.partial(
        pl.kernel,
        mesh=mesh,
        out_type=jax.ShapeDtypeStruct((n_total, D_MODEL), jnp.float32),
        scratch_types=[
            pltpu.VMEM((n_sub, SUB), jnp.int32),
            pltpu.VMEM((NBUF, SUB, D_MODEL), jnp.float32),
            pltpu.VMEM_SHARED((366, D_MODEL), jnp.float32),
            pltpu.SemaphoreType.DMA,
            pltpu.SemaphoreType.DMA,
            pltpu.SemaphoreType.DMA,
            pltpu.SemaphoreType.DMA,
            pltpu.SemaphoreType.DMA,
            pltpu.SemaphoreType.DMA,
            pltpu.SemaphoreType.DMA,
            pltpu.SemaphoreType.DMA,
        ],
    )
    def gather_kernel(idx_hbm, table_hbm, out_hbm, idx_v, rows, table_sh,
                      g0, g1, g2, g3, s0, s1, s2, s3):
        wid = lax.axis_index("s") * info.num_cores + lax.axis_index("c")
        base = wid * n_per_w

        @pl.when(lax.axis_index("s") == 0)
        def _():
            pltpu.sync_copy(table_hbm, table_sh)
        plsc.subcore_barrier()

        pltpu.sync_copy(idx_hbm.at[wid], idx_v)

        gsems = (g0, g1, g2, g3)
        ssems = (s0, s1, s2, s3)

        def body(io, carry):
            for b in range(NBUF):
                i = io * NBUF + b

                @pl.when(io > 0)
                def _():
                    pltpu.make_async_copy(
                        rows.at[b], out_hbm.at[pl.ds(base, SUB)],
                        ssems[b]).wait()

                pltpu.async_copy(
                    table_sh.at[idx_v.at[i]], rows.at[b], gsems[b])
            for b in range(NBUF):
                i = io * NBUF + b
                pltpu.make_async_copy(
                    table_sh.at[idx_v.at[i]], rows.at[b], gsems[b]).wait()
                pltpu.async_copy(
                    rows.at[b], out_hbm.at[pl.ds(base + i * SUB, SUB)],
                    ssems[b])
            return carry

        lax.fori_loop(0, n_outer, body, 0)

        for b in range(NBUF):
            pltpu.make_async_copy(
                rows.at[b], out_hbm.at[pl.ds(base, SUB)], ssems[b]).wait()

    return gather_kernel


def kernel(positions, pe):
    b, t = positions.shape
    n_total = b * t
    info = plsc.get_sparse_core_info()
    nw = info.num_cores * info.num_subcores
    idx = positions.reshape(nw, (n_total // nw) // SUB, SUB).astype(jnp.int32)
    out = _gather_sc(n_total)(idx, pe)
    return out.reshape(b, t, D_MODEL)
